# Initial kernel scaffold; baseline (speedup 1.0000x reference)
import jax
import jax.numpy as jnp
from jax.experimental import pallas as pl


def kernel(x, table):
    raise NotImplementedError("write your pallas kernel here")


# speedup vs baseline: 10.0137x; 10.0137x over previous
import functools

import jax
import jax.numpy as jnp
from jax import lax
from jax.experimental import pallas as pl
from jax.experimental.pallas import tpu as pltpu
from jax.experimental.pallas import tpu_sc as plsc

B = 4096
N = 20
S = 20
E = 32
R = B * S

NC, NS = 2, 16
NW = NC * NS
RPW = R // NW
CK = 64
GPC = CK * N // 128
NCHUNK = RPW // CK


def _sc_body(idx_hbm, table_hbm, out_hbm, idx_v, rows_v, out_v, sem):
    wid = lax.axis_index("s") * NC + lax.axis_index("c")
    wrow = wid * RPW

    def acc_body(k, carry):
        base = k * N
        a0 = rows_v[base, pl.ds(0, 16)]
        a1 = rows_v[base, pl.ds(16, 16)]
        for n in range(1, N):
            a0 = a0 + rows_v[base + n, pl.ds(0, 16)]
            a1 = a1 + rows_v[base + n, pl.ds(16, 16)]
        out_v[k, pl.ds(0, 16)] = a0
        out_v[k, pl.ds(16, 16)] = a1
        return carry

    def chunk_body(c, carry):
        i0 = (wrow + c * CK) * N
        pltpu.sync_copy(idx_hbm.at[pl.ds(i0, CK * N)], idx_v)
        descs = [
            pltpu.async_copy(
                table_hbm.at[idx_v.at[pl.ds(g * 128, 128)]],
                rows_v.at[pl.ds(g * 128, 128), :],
                sem,
            )
            for g in range(GPC)
        ]
        for d in descs:
            d.wait()
        lax.fori_loop(0, CK, acc_body, 0)
        pltpu.sync_copy(out_v, out_hbm.at[pl.ds(wrow + c * CK, CK), :])
        return carry

    lax.fori_loop(0, NCHUNK, chunk_body, 0)


@functools.cache
def _sc_call():
    return functools.partial(
        pl.kernel,
        out_type=jax.ShapeDtypeStruct((R, E), jnp.float32),
        mesh=plsc.VectorSubcoreMesh(
            core_axis_name="c", subcore_axis_name="s",
            num_cores=NC, num_subcores=NS,
        ),
        scratch_types=[
            pltpu.VMEM((CK * N,), jnp.int32),
            pltpu.VMEM((CK * N, E), jnp.float32),
            pltpu.VMEM((CK, E), jnp.float32),
            pltpu.SemaphoreType.DMA,
        ],
        compiler_params=pltpu.CompilerParams(use_tc_tiling_on_sc=False),
    )(_sc_body)


def kernel(x, table):
    xt = jnp.swapaxes(x, 1, 2).reshape(R * N)
    out = _sc_call()(xt, table)
    return out.reshape(B, 1, S, E)


# speedup vs baseline: 10.2951x; 1.0281x over previous
import functools

import jax
import jax.numpy as jnp
from jax import lax
from jax.experimental import pallas as pl
from jax.experimental.pallas import tpu as pltpu
from jax.experimental.pallas import tpu_sc as plsc

B = 4096
N = 20
S = 20
E = 32
R = B * S

NC, NS = 2, 16
NW = NC * NS
BPW = B // NW
CB = 4
CI = CB * N * S
CR = CB * S
GSZ = 80
GPC = CI // GSZ
NCHUNK = BPW // CB


def _sc_body(idx_hbm, table_hbm, out_hbm, idx_v, rows_v, out_v, sem):
    wid = lax.axis_index("s") * NC + lax.axis_index("c")
    wb = wid * BPW

    def acc_body(r, carry):
        bb = r // S
        s = r - bb * S
        base = bb * (N * S) + s
        a0 = rows_v[base, pl.ds(0, 16)]
        a1 = rows_v[base, pl.ds(16, 16)]
        for n in range(1, N):
            a0 = a0 + rows_v[base + n * S, pl.ds(0, 16)]
            a1 = a1 + rows_v[base + n * S, pl.ds(16, 16)]
        out_v[r, pl.ds(0, 16)] = a0
        out_v[r, pl.ds(16, 16)] = a1
        return carry

    def chunk_body(c, carry):
        b0 = wb + c * CB
        pltpu.sync_copy(idx_hbm.at[pl.ds(b0 * N * S, CI)], idx_v)
        descs = [
            pltpu.async_copy(
                table_hbm.at[idx_v.at[pl.ds(g * GSZ, GSZ)]],
                rows_v.at[pl.ds(g * GSZ, GSZ), :],
                sem,
            )
            for g in range(GPC)
        ]
        for d in descs:
            d.wait()
        lax.fori_loop(0, CR, acc_body, 0)
        pltpu.sync_copy(out_v, out_hbm.at[pl.ds(b0 * S, CR), :])
        return carry

    lax.fori_loop(0, NCHUNK, chunk_body, 0)


@functools.cache
def _sc_call():
    return functools.partial(
        pl.kernel,
        out_type=jax.ShapeDtypeStruct((R, E), jnp.float32),
        mesh=plsc.VectorSubcoreMesh(
            core_axis_name="c", subcore_axis_name="s",
            num_cores=NC, num_subcores=NS,
        ),
        scratch_types=[
            pltpu.VMEM((CI,), jnp.int32),
            pltpu.VMEM((CI, E), jnp.float32),
            pltpu.VMEM((CR, E), jnp.float32),
            pltpu.SemaphoreType.DMA,
        ],
        compiler_params=pltpu.CompilerParams(use_tc_tiling_on_sc=False),
    )(_sc_body)


def kernel(x, table):
    out = _sc_call()(x.reshape(B * N * S), table)
    return out.reshape(B, 1, S, E)


# speedup vs baseline: 10.4161x; 1.0118x over previous
import functools

import jax
import jax.numpy as jnp
from jax import lax
from jax.experimental import pallas as pl
from jax.experimental.pallas import tpu as pltpu
from jax.experimental.pallas import tpu_sc as plsc

B = 4096
N = 20
S = 20
E = 32

NC, NS = 2, 16
NW = NC * NS
BPW = B // NW
CB = 4
CI = CB * N * S
CR = CB * S
GSZ = 80
GPC = CI // GSZ
NCHUNK = BPW // CB


def _sc_body(idx_hbm, table_hbm, out_hbm, idx_v, rows_v, out_v, sem):
    wid = lax.axis_index("s") * NC + lax.axis_index("c")
    wb = wid * BPW

    def acc_body(r, carry):
        bb = r // S
        s = r - bb * S
        base = bb * (N * S) + s
        a0 = rows_v[base, pl.ds(0, 16)]
        a1 = rows_v[base, pl.ds(16, 16)]
        for n in range(1, N):
            a0 = a0 + rows_v[base + n * S, pl.ds(0, 16)]
            a1 = a1 + rows_v[base + n * S, pl.ds(16, 16)]
        out_v[bb, s, pl.ds(0, 16)] = a0
        out_v[bb, s, pl.ds(16, 16)] = a1
        return carry

    def chunk_body(c, carry):
        b0 = wb + c * CB
        pltpu.sync_copy(idx_hbm.at[pl.ds(b0, CB), :], idx_v)
        descs = [
            pltpu.async_copy(
                table_hbm.at[idx_v.at[bb, pl.ds(g * GSZ, GSZ)]],
                rows_v.at[pl.ds(bb * (N * S) + g * GSZ, GSZ), :],
                sem,
            )
            for bb in range(CB)
            for g in range(N * S // GSZ)
        ]
        for d in descs:
            d.wait()
        lax.fori_loop(0, CR, acc_body, 0)
        pltpu.sync_copy(out_v, out_hbm.at[pl.ds(b0, CB), 0, :, :])
        return carry

    lax.fori_loop(0, NCHUNK, chunk_body, 0)


@functools.cache
def _sc_call():
    return functools.partial(
        pl.kernel,
        out_type=jax.ShapeDtypeStruct((B, 1, S, E), jnp.float32),
        mesh=plsc.VectorSubcoreMesh(
            core_axis_name="c", subcore_axis_name="s",
            num_cores=NC, num_subcores=NS,
        ),
        scratch_types=[
            pltpu.VMEM((CB, N * S), jnp.int32),
            pltpu.VMEM((CI, E), jnp.float32),
            pltpu.VMEM((CB, S, E), jnp.float32),
            pltpu.SemaphoreType.DMA,
        ],
        compiler_params=pltpu.CompilerParams(use_tc_tiling_on_sc=False),
    )(_sc_body)


def kernel(x, table):
    return _sc_call()(x.reshape(B, N * S), table)
